# baseline (speedup 1.0000x reference)
import functools
import jax
import jax.numpy as jnp
from jax import lax
from jax.experimental import pallas as pl
from jax.experimental.pallas import tpu as pltpu
from jax.experimental.pallas import tpu_sc as plsc

_K = 8
_B, _S, _C = 4, 8192, 1024
_CHUNK = 512
_CB = 128


def _sc_body(x_hbm, o_hbm, buf, outb):
    wid = lax.axis_index("s") * 2 + lax.axis_index("c")
    gid = wid
    b = gid // (_C // _CB)
    c0 = (gid % (_C // _CB)) * _CB
    neg = jnp.float32(-jnp.inf)

    for k in range(_CB // 16):
        for j in range(_K):
            outb[j, pl.ds(k * 16, 16)] = jnp.full((16,), neg, jnp.float32)

    for ch in range(_S // _CHUNK):
        pltpu.sync_copy(
            x_hbm.at[b, pl.ds(ch * _CHUNK, _CHUNK), pl.ds(c0, _CB)], buf)

        def row_body(r, _):
            for k in range(_CB // 16):
                v = buf[r, pl.ds(k * 16, 16)]
                T = [outb[j, pl.ds(k * 16, 16)] for j in range(_K)]
                for j in range(_K):
                    hi = jnp.maximum(T[j], v)
                    if j < _K - 1:
                        v = jnp.minimum(T[j], v)
                    outb[j, pl.ds(k * 16, 16)] = hi
            return 0

        lax.fori_loop(0, _CHUNK, row_body, 0)

    pltpu.sync_copy(outb, o_hbm.at[b, :, pl.ds(c0, _CB)])


def kernel(inputs):
    out3 = functools.partial(
        pl.kernel,
        mesh=plsc.VectorSubcoreMesh(core_axis_name="c", subcore_axis_name="s"),
        out_type=jax.ShapeDtypeStruct((_B, _K, _C), jnp.float32),
        scratch_types=[
            pltpu.VMEM((_CHUNK, _CB), jnp.float32),
            pltpu.VMEM((_K, _CB), jnp.float32),
        ],
    )(_sc_body)(inputs)
    return jnp.transpose(out3, (0, 2, 1)).reshape(_B, _C * _K)
